# baseline (speedup 1.0000x reference)
import functools

import numpy as np

import jax
import jax.numpy as jnp
from jax import lax
from jax.experimental import pallas as pl
from jax.experimental.pallas import tpu as pltpu
from jax.experimental.pallas import tpu_sc as plsc

N = 10000
NP = 10240
DF = 128
DC = 64
NCLS = 40
E = 320000
B = 128
NTILES = 16
NCORES = 2
EPT = 10240
EPAD = EPT * NTILES * NCORES
NBATCH = EPT // B
NB_A = 80
NB_B = 2 * NBATCH - NB_A
PAD_IDX = N
RPT = NP // NTILES


_MESH = plsc.VectorSubcoreMesh(
    core_axis_name="c", subcore_axis_name="s",
    num_cores=NCORES, num_subcores=NTILES)
_SC_PARAMS = pltpu.CompilerParams(use_tc_tiling_on_sc=False)


def _make_agg(d):
    @functools.partial(
        pl.kernel,
        out_type=(jax.ShapeDtypeStruct((NP, d), jnp.float32),
                  jax.ShapeDtypeStruct((NP, d), jnp.float32)),
        mesh=_MESH,
        scratch_types=[
            pltpu.VMEM((NB_B, B), jnp.int32),
            pltpu.VMEM((NB_B, B), jnp.int32),
            pltpu.VMEM((B, d), jnp.float32),
            pltpu.VMEM((B, d), jnp.float32),
            pltpu.VMEM((B, d), jnp.float32),
            pltpu.VMEM((B, d), jnp.float32),
            pltpu.VMEM_SHARED((NP, d), jnp.float32),
            pltpu.SemaphoreType.DMA,
            pltpu.SemaphoreType.DMA,
            pltpu.SemaphoreType.DMA,
            pltpu.SemaphoreType.DMA,
            pltpu.SemaphoreType.DMA,
            pltpu.SemaphoreType.DMA,
            pltpu.SemaphoreType.DMA,
            pltpu.SemaphoreType.DMA,
        ],
        compiler_params=_SC_PARAMS,
    )
    def agg(g_hbm, src2_hbm, dst2_hbm, zero_hbm, out_a, out_b,
            srcs_v, dsts_v, r0, r1, r2, r3, acc_sh,
            sg0, sg1, sg2, sg3, ss0, ss1, ss2, ss3):
        rows = (r0, r1, r2, r3)
        sg = (sg0, sg1, sg2, sg3)
        ss = (ss0, ss1, ss2, ss3)
        c = lax.axis_index("c")
        s = lax.axis_index("s")
        row0 = s * RPT

        pltpu.sync_copy(zero_hbm.at[pl.ds(row0, RPT)],
                        acc_sh.at[pl.ds(row0, RPT)])

        @pl.when(c == 0)
        def _():
            tb = s * NB_A
            pltpu.sync_copy(src2_hbm.at[pl.ds(tb, NB_A)],
                            srcs_v.at[pl.ds(0, NB_A)])
            pltpu.sync_copy(dst2_hbm.at[pl.ds(tb, NB_A)],
                            dsts_v.at[pl.ds(0, NB_A)])

        @pl.when(c == 1)
        def _():
            tb = NTILES * NB_A + s * NB_B
            pltpu.sync_copy(src2_hbm.at[pl.ds(tb, NB_B)], srcs_v)
            pltpu.sync_copy(dst2_hbm.at[pl.ds(tb, NB_B)], dsts_v)

        nb4 = jnp.where(c == 0, NB_A // 4, NB_B // 4)
        plsc.subcore_barrier()

        def gather(i, p):
            return pltpu.async_copy(g_hbm.at[srcs_v.at[i]], rows[p], sg[p])

        def gather_wait(i, p):
            pltpu.make_async_copy(g_hbm.at[srcs_v.at[i]], rows[p],
                                  sg[p]).wait()

        def scatter(i, p):
            return pltpu.async_copy(rows[p], acc_sh.at[dsts_v.at[i]],
                                    ss[p], add=True)

        def scatter_wait(i, p):
            pltpu.make_async_copy(rows[p], acc_sh.at[dsts_v.at[i]],
                                  ss[p]).wait()

        def body(j, carry):
            for p in range(4):
                i = 4 * j + p

                @pl.when(j > 0)
                def _(i=i, p=p):
                    scatter_wait(i - 4, p)

                gather(i, p)
            for p in range(4):
                i = 4 * j + p
                gather_wait(i, p)
                scatter(i, p)
            return carry

        lax.fori_loop(0, nb4, body, 0)
        for p in range(4):
            scatter_wait((nb4 - 1) * 4 + p, p)
        plsc.subcore_barrier()

        @pl.when(c == 0)
        def _():
            pltpu.sync_copy(acc_sh.at[pl.ds(row0, RPT)],
                            out_a.at[pl.ds(row0, RPT)])

        @pl.when(c == 1)
        def _():
            pltpu.sync_copy(acc_sh.at[pl.ds(row0, RPT)],
                            out_b.at[pl.ds(row0, RPT)])

    return agg


@functools.partial(
    pl.kernel,
    out_type=(jax.ShapeDtypeStruct((NP, 16), jnp.float32),
              jax.ShapeDtypeStruct((NP, 16), jnp.float32)),
    mesh=_MESH,
    scratch_types=[
        pltpu.VMEM((NBATCH, B), jnp.int32),
        pltpu.VMEM((B, 16), jnp.float32),
        pltpu.VMEM_SHARED((NP, 16), jnp.float32),
        pltpu.SemaphoreType.DMA,
    ],
    compiler_params=_SC_PARAMS,
)
def _deg(ones_hbm, dst2_hbm, zero_hbm, out_a, out_b,
         dsts_v, ones_v, acc_sh, sem):
    c = lax.axis_index("c")
    s = lax.axis_index("s")
    row0 = s * RPT
    W = 8

    pltpu.sync_copy(zero_hbm.at[pl.ds(row0, RPT)],
                    acc_sh.at[pl.ds(row0, RPT)])

    pltpu.sync_copy(ones_hbm.at[pl.ds(0, B)], ones_v)
    tb = (c * NTILES + s) * NBATCH
    pltpu.sync_copy(dst2_hbm.at[pl.ds(tb, NBATCH)], dsts_v)
    plsc.subcore_barrier()

    def body(i, carry):
        pltpu.async_copy(ones_v, acc_sh.at[dsts_v.at[i]], sem, add=True)

        @pl.when(i >= W)
        def _():
            pltpu.make_async_copy(ones_v, acc_sh.at[dsts_v.at[i - W]],
                                  sem).wait()
        return carry

    lax.fori_loop(0, NBATCH, body, 0)

    def drain(i, carry):
        pltpu.make_async_copy(ones_v, acc_sh.at[dsts_v.at[NBATCH - W + i]],
                              sem).wait()
        return carry

    lax.fori_loop(0, W, drain, 0)
    plsc.subcore_barrier()

    @pl.when(c == 0)
    def _():
        pltpu.sync_copy(acc_sh.at[pl.ds(row0, RPT)],
                        out_a.at[pl.ds(row0, RPT)])

    @pl.when(c == 1)
    def _():
        pltpu.sync_copy(acc_sh.at[pl.ds(row0, RPT)],
                        out_b.at[pl.ds(row0, RPT)])


_agg64 = _make_agg(DC)

_EPT2 = EPAD // NTILES
_NBATCH2 = _EPT2 // B


@functools.partial(
    pl.kernel,
    out_type=(jax.ShapeDtypeStruct((NP, 64), jnp.float32),
              jax.ShapeDtypeStruct((NP, 64), jnp.float32)),
    mesh=_MESH,
    scratch_types=[
        pltpu.VMEM((_NBATCH2, B), jnp.int32),
        pltpu.VMEM((_NBATCH2, B), jnp.int32),
        pltpu.VMEM((B, 64), jnp.float32),
        pltpu.VMEM((B, 64), jnp.float32),
        pltpu.VMEM((B, 64), jnp.float32),
        pltpu.VMEM((B, 64), jnp.float32),
        pltpu.VMEM_SHARED((NP, 64), jnp.float32),
        pltpu.SemaphoreType.DMA,
        pltpu.SemaphoreType.DMA,
        pltpu.SemaphoreType.DMA,
        pltpu.SemaphoreType.DMA,
        pltpu.SemaphoreType.DMA,
        pltpu.SemaphoreType.DMA,
        pltpu.SemaphoreType.DMA,
        pltpu.SemaphoreType.DMA,
    ],
    compiler_params=_SC_PARAMS,
)
def _agg_l0(glo_hbm, ghi_hbm, src2_hbm, dst2_hbm, zero_hbm, out_lo, out_hi,
            srcs_v, dsts_v, r0, r1, r2, r3, acc_sh, sg0, sg1, sg2, sg3,
            ss0, ss1, ss2, ss3):
    c = lax.axis_index("c")
    s = lax.axis_index("s")
    rows = (r0, r1, r2, r3)
    sg = (sg0, sg1, sg2, sg3)
    ss = (ss0, ss1, ss2, ss3)
    row0 = s * RPT
    tb = s * _NBATCH2

    def run(gref, outref):
        pltpu.sync_copy(zero_hbm.at[pl.ds(row0, RPT)],
                        acc_sh.at[pl.ds(row0, RPT)])
        pltpu.sync_copy(src2_hbm.at[pl.ds(tb, _NBATCH2)], srcs_v)
        pltpu.sync_copy(dst2_hbm.at[pl.ds(tb, _NBATCH2)], dsts_v)
        plsc.subcore_barrier()

        def body(j, carry):
            for p in range(4):
                i = 4 * j + p

                @pl.when(j > 0)
                def _(i=i, p=p):
                    pltpu.make_async_copy(
                        rows[p], acc_sh.at[dsts_v.at[i - 4]], ss[p]).wait()

                pltpu.async_copy(gref.at[srcs_v.at[i]], rows[p], sg[p])
            for p in range(4):
                i = 4 * j + p
                pltpu.make_async_copy(gref.at[srcs_v.at[i]], rows[p],
                                      sg[p]).wait()
                pltpu.async_copy(rows[p], acc_sh.at[dsts_v.at[i]],
                                 ss[p], add=True)
            return carry

        lax.fori_loop(0, _NBATCH2 // 4, body, 0)
        for p in range(4):
            pltpu.make_async_copy(
                rows[p], acc_sh.at[dsts_v.at[_NBATCH2 - 4 + p]],
                ss[p]).wait()
        plsc.subcore_barrier()
        pltpu.sync_copy(acc_sh.at[pl.ds(row0, RPT)],
                        outref.at[pl.ds(row0, RPT)])

    @pl.when(c == 0)
    def _():
        run(glo_hbm, out_lo)

    @pl.when(c == 1)
    def _():
        run(ghi_hbm, out_hi)

_RBLK = 1024
_GRID = NP // _RBLK


def _tc1_body(ha_ref, hb_ref, x_ref, glo_ref, ghi_ref, dinv_ref):
    i = pl.program_id(0)
    rows = lax.broadcasted_iota(jnp.int32, (_RBLK, 1), 0) + i * _RBLK
    deg = ha_ref[:, 0:1] + hb_ref[:, 0:1] + 1.0
    dinv = jnp.where(rows < N, lax.rsqrt(deg), 0.0)
    dinv_ref[...] = dinv
    g = x_ref[...] * dinv
    glo_ref[...] = g[:, :64]
    ghi_ref[...] = g[:, 64:]


def _tc1(ha, hb, x_pad):
    return pl.pallas_call(
        _tc1_body,
        grid=(_GRID,),
        in_specs=[
            pl.BlockSpec((_RBLK, 16), lambda i: (i, 0)),
            pl.BlockSpec((_RBLK, 16), lambda i: (i, 0)),
            pl.BlockSpec((_RBLK, DF), lambda i: (i, 0)),
        ],
        out_specs=[
            pl.BlockSpec((_RBLK, 64), lambda i: (i, 0)),
            pl.BlockSpec((_RBLK, 64), lambda i: (i, 0)),
            pl.BlockSpec((_RBLK, 1), lambda i: (i, 0)),
        ],
        out_shape=[
            jax.ShapeDtypeStruct((NP, 64), jnp.float32),
            jax.ShapeDtypeStruct((NP, 64), jnp.float32),
            jax.ShapeDtypeStruct((NP, 1), jnp.float32),
        ],
    )(ha, hb, x_pad)


def _tc2_body(a_ref, b_ref, glo_ref, ghi_ref, dinv_ref, w0_ref, b0_ref,
              w1_ref, q_ref):
    dinv = dinv_ref[...]
    a0 = (jnp.concatenate([a_ref[...], b_ref[...]], axis=1)
          + jnp.concatenate([glo_ref[...], ghi_ref[...]], axis=1)) * dinv
    h = jnp.dot(a0, w0_ref[...], preferred_element_type=jnp.float32)
    h = jnp.maximum(h + b0_ref[...], 0.0)
    q = jnp.dot(h, w1_ref[...], preferred_element_type=jnp.float32)
    q_ref[...] = q * dinv


def _tc2(acc_a, acc_b, glo, ghi, dinv, W0, b0r, W1p):
    return pl.pallas_call(
        _tc2_body,
        grid=(_GRID,),
        in_specs=[
            pl.BlockSpec((_RBLK, 64), lambda i: (i, 0)),
            pl.BlockSpec((_RBLK, 64), lambda i: (i, 0)),
            pl.BlockSpec((_RBLK, 64), lambda i: (i, 0)),
            pl.BlockSpec((_RBLK, 64), lambda i: (i, 0)),
            pl.BlockSpec((_RBLK, 1), lambda i: (i, 0)),
            pl.BlockSpec((DF, DF), lambda i: (0, 0)),
            pl.BlockSpec((1, DF), lambda i: (0, 0)),
            pl.BlockSpec((DF, DC), lambda i: (0, 0)),
        ],
        out_specs=pl.BlockSpec((_RBLK, DC), lambda i: (i, 0)),
        out_shape=jax.ShapeDtypeStruct((NP, DC), jnp.float32),
    )(acc_a, acc_b, glo, ghi, dinv, W0, b0r, W1p)


def _tc3_body(a_ref, b_ref, q_ref, dinv_ref, b1_ref, out_ref):
    z = ((a_ref[...] + b_ref[...] + q_ref[...]) * dinv_ref[...]
         + b1_ref[...])
    col = lax.broadcasted_iota(jnp.int32, (_RBLK, DC), 1)
    valid = col < NCLS
    zm = jnp.where(valid, z, -jnp.inf)
    m = jnp.max(zm, axis=1, keepdims=True)
    lse = jnp.log(jnp.sum(jnp.where(valid, jnp.exp(z - m), 0.0),
                          axis=1, keepdims=True))
    out_ref[...] = (z - m - lse)[:, :NCLS]


def _tc3(acc_a, acc_b, q1, dinv, b1r):
    return pl.pallas_call(
        _tc3_body,
        grid=(_GRID,),
        in_specs=[
            pl.BlockSpec((_RBLK, DC), lambda i: (i, 0)),
            pl.BlockSpec((_RBLK, DC), lambda i: (i, 0)),
            pl.BlockSpec((_RBLK, DC), lambda i: (i, 0)),
            pl.BlockSpec((_RBLK, 1), lambda i: (i, 0)),
            pl.BlockSpec((1, DC), lambda i: (0, 0)),
        ],
        out_specs=pl.BlockSpec((_RBLK, NCLS), lambda i: (i, 0)),
        out_shape=jax.ShapeDtypeStruct((NP, NCLS), jnp.float32),
    )(acc_a, acc_b, q1, dinv, b1r)


def kernel(x, edge_index, W0, b0, W1, b1):
    pad = jnp.asarray(N + (np.arange(EPAD - E) % (NP - N)).astype(np.int32))
    src = jnp.concatenate([edge_index[0], pad]).reshape(EPAD // B, B)
    dst = jnp.concatenate([edge_index[1], pad]).reshape(EPAD // B, B)
    x_pad = jnp.zeros((NP, DF), jnp.float32).at[:N].set(x)
    ones16 = jnp.ones((NP, 16), jnp.float32)
    z16 = jnp.zeros((NP, 16), jnp.float32)
    z64 = jnp.zeros((NP, DC), jnp.float32)
    W1p = jnp.zeros((DF, DC), jnp.float32).at[:, :NCLS].set(W1)
    b1r = jnp.zeros((1, DC), jnp.float32).at[0, :NCLS].set(b1)

    ha, hb = _deg(ones16, dst, z16)
    glo, ghi, dinv = _tc1(ha, hb, x_pad)
    s0lo, s0hi = _agg_l0(glo, ghi, src, dst, z64)
    q1 = _tc2(s0lo, s0hi, glo, ghi, dinv, W0, b0.reshape(1, DF), W1p)
    a1a, a1b = _agg64(q1, src, dst, z64)
    out_full = _tc3(a1a, a1b, q1, dinv, b1r)
    return out_full[:N]


# speedup vs baseline: 1.0320x; 1.0320x over previous
import functools

import numpy as np

import jax
import jax.numpy as jnp
from jax import lax
from jax.experimental import pallas as pl
from jax.experimental.pallas import tpu as pltpu
from jax.experimental.pallas import tpu_sc as plsc

N = 10000
NP = 10240
DF = 128
DC = 64
NCLS = 40
E = 320000
B = 128
NTILES = 16
NCORES = 2
EPAD = 327680
NBATCH = EPAD // B // (NTILES * NCORES)
RBATCH = E // B
PADB = EPAD // B - RBATCH
LASTR = RBATCH - (NTILES * NCORES - 1) * NBATCH
RPT = NP // NTILES

_MESH = plsc.VectorSubcoreMesh(
    core_axis_name="c", subcore_axis_name="s",
    num_cores=NCORES, num_subcores=NTILES)
_SC_PARAMS = pltpu.CompilerParams(use_tc_tiling_on_sc=False)


def _make_agg(d):
    @functools.partial(
        pl.kernel,
        out_type=(jax.ShapeDtypeStruct((NP, d), jnp.float32),
                  jax.ShapeDtypeStruct((NP, d), jnp.float32)),
        mesh=_MESH,
        scratch_types=[
            pltpu.VMEM((NBATCH * B,), jnp.int32),
            pltpu.VMEM((NBATCH * B,), jnp.int32),
            pltpu.VMEM((B, d), jnp.float32),
            pltpu.VMEM((B, d), jnp.float32),
            pltpu.VMEM((B, d), jnp.float32),
            pltpu.VMEM((B, d), jnp.float32),
            pltpu.VMEM_SHARED((NP, d), jnp.float32),
            pltpu.SemaphoreType.DMA,
            pltpu.SemaphoreType.DMA,
            pltpu.SemaphoreType.DMA,
            pltpu.SemaphoreType.DMA,
            pltpu.SemaphoreType.DMA,
            pltpu.SemaphoreType.DMA,
            pltpu.SemaphoreType.DMA,
            pltpu.SemaphoreType.DMA,
        ],
        compiler_params=_SC_PARAMS,
    )
    def agg(g_hbm, eflat_hbm, padc_hbm, zero_hbm, out_a, out_b,
            srcs_v, dsts_v, r0, r1, r2, r3, acc_sh,
            sg0, sg1, sg2, sg3, ss0, ss1, ss2, ss3):
        rows = (r0, r1, r2, r3)
        sg = (sg0, sg1, sg2, sg3)
        ss = (ss0, ss1, ss2, ss3)
        c = lax.axis_index("c")
        s = lax.axis_index("s")
        row0 = s * RPT

        pltpu.sync_copy(zero_hbm.at[pl.ds(row0, RPT)],
                        acc_sh.at[pl.ds(row0, RPT)])

        t = c * NTILES + s
        eb = t * NBATCH * B

        @pl.when(t < NTILES * NCORES - 1)
        def _():
            pltpu.sync_copy(eflat_hbm.at[pl.ds(eb, NBATCH * B)], srcs_v)
            pltpu.sync_copy(eflat_hbm.at[pl.ds(E + eb, NBATCH * B)], dsts_v)

        @pl.when(t == NTILES * NCORES - 1)
        def _():
            pltpu.sync_copy(eflat_hbm.at[pl.ds(eb, LASTR * B)],
                            srcs_v.at[pl.ds(0, LASTR * B)])
            pltpu.sync_copy(padc_hbm.at[pl.ds(0, PADB * B)],
                            srcs_v.at[pl.ds(LASTR * B, PADB * B)])
            pltpu.sync_copy(eflat_hbm.at[pl.ds(E + eb, LASTR * B)],
                            dsts_v.at[pl.ds(0, LASTR * B)])
            pltpu.sync_copy(padc_hbm.at[pl.ds(0, PADB * B)],
                            dsts_v.at[pl.ds(LASTR * B, PADB * B)])

        plsc.subcore_barrier()

        def gather(i, p):
            return pltpu.async_copy(
                g_hbm.at[srcs_v.at[pl.ds(i * B, B)]], rows[p], sg[p])

        def gather_wait(i, p):
            pltpu.make_async_copy(
                g_hbm.at[srcs_v.at[pl.ds(i * B, B)]], rows[p], sg[p]).wait()

        def scatter(i, p):
            return pltpu.async_copy(
                rows[p], acc_sh.at[dsts_v.at[pl.ds(i * B, B)]],
                ss[p], add=True)

        def scatter_wait(i, p):
            pltpu.make_async_copy(
                rows[p], acc_sh.at[dsts_v.at[pl.ds(i * B, B)]],
                ss[p]).wait()

        def body(j, carry):
            for p in range(4):
                i = 4 * j + p

                @pl.when(j > 0)
                def _(i=i, p=p):
                    scatter_wait(i - 4, p)

                gather(i, p)
            for p in range(4):
                i = 4 * j + p
                gather_wait(i, p)
                scatter(i, p)
            return carry

        lax.fori_loop(0, NBATCH // 4, body, 0)
        for p in range(4):
            scatter_wait(NBATCH - 4 + p, p)
        plsc.subcore_barrier()

        @pl.when(c == 0)
        def _():
            pltpu.sync_copy(acc_sh.at[pl.ds(row0, RPT)],
                            out_a.at[pl.ds(row0, RPT)])

        @pl.when(c == 1)
        def _():
            pltpu.sync_copy(acc_sh.at[pl.ds(row0, RPT)],
                            out_b.at[pl.ds(row0, RPT)])

    return agg


@functools.partial(
    pl.kernel,
    out_type=(jax.ShapeDtypeStruct((NP, 16), jnp.float32),
              jax.ShapeDtypeStruct((NP, 16), jnp.float32)),
    mesh=_MESH,
    scratch_types=[
        pltpu.VMEM((NBATCH * B,), jnp.int32),
        pltpu.VMEM((B, 16), jnp.float32),
        pltpu.VMEM_SHARED((NP, 16), jnp.float32),
        pltpu.SemaphoreType.DMA,
    ],
    compiler_params=_SC_PARAMS,
)
def _deg(ones_hbm, eflat_hbm, padc_hbm, zero_hbm, out_a, out_b,
         dsts_v, ones_v, acc_sh, sem):
    c = lax.axis_index("c")
    s = lax.axis_index("s")
    row0 = s * RPT
    W = 8

    pltpu.sync_copy(zero_hbm.at[pl.ds(row0, RPT)],
                    acc_sh.at[pl.ds(row0, RPT)])
    pltpu.sync_copy(ones_hbm, ones_v)

    t = c * NTILES + s
    eb = t * NBATCH * B

    @pl.when(t < NTILES * NCORES - 1)
    def _():
        pltpu.sync_copy(eflat_hbm.at[pl.ds(E + eb, NBATCH * B)], dsts_v)

    @pl.when(t == NTILES * NCORES - 1)
    def _():
        pltpu.sync_copy(eflat_hbm.at[pl.ds(E + eb, LASTR * B)],
                        dsts_v.at[pl.ds(0, LASTR * B)])
        pltpu.sync_copy(padc_hbm.at[pl.ds(0, PADB * B)],
                        dsts_v.at[pl.ds(LASTR * B, PADB * B)])

    plsc.subcore_barrier()

    def body(i, carry):
        pltpu.async_copy(ones_v, acc_sh.at[dsts_v.at[pl.ds(i * B, B)]],
                         sem, add=True)

        @pl.when(i >= W)
        def _():
            pltpu.make_async_copy(
                ones_v, acc_sh.at[dsts_v.at[pl.ds((i - W) * B, B)]],
                sem).wait()
        return carry

    lax.fori_loop(0, NBATCH, body, 0)

    def drain(i, carry):
        pltpu.make_async_copy(
            ones_v, acc_sh.at[dsts_v.at[pl.ds((NBATCH - W + i) * B, B)]],
            sem).wait()
        return carry

    lax.fori_loop(0, W, drain, 0)
    plsc.subcore_barrier()

    @pl.when(c == 0)
    def _():
        pltpu.sync_copy(acc_sh.at[pl.ds(row0, RPT)],
                        out_a.at[pl.ds(row0, RPT)])

    @pl.when(c == 1)
    def _():
        pltpu.sync_copy(acc_sh.at[pl.ds(row0, RPT)],
                        out_b.at[pl.ds(row0, RPT)])


_agg64 = _make_agg(DC)

_NBATCH2 = EPAD // B // NTILES
_LASTR2 = RBATCH - (NTILES - 1) * _NBATCH2


@functools.partial(
    pl.kernel,
    out_type=(jax.ShapeDtypeStruct((NP, 64), jnp.float32),
              jax.ShapeDtypeStruct((NP, 64), jnp.float32)),
    mesh=_MESH,
    scratch_types=[
        pltpu.VMEM((_NBATCH2 * B,), jnp.int32),
        pltpu.VMEM((_NBATCH2 * B,), jnp.int32),
        pltpu.VMEM((B, 64), jnp.float32),
        pltpu.VMEM((B, 64), jnp.float32),
        pltpu.VMEM((B, 64), jnp.float32),
        pltpu.VMEM((B, 64), jnp.float32),
        pltpu.VMEM_SHARED((NP, 64), jnp.float32),
        pltpu.SemaphoreType.DMA,
        pltpu.SemaphoreType.DMA,
        pltpu.SemaphoreType.DMA,
        pltpu.SemaphoreType.DMA,
        pltpu.SemaphoreType.DMA,
        pltpu.SemaphoreType.DMA,
        pltpu.SemaphoreType.DMA,
        pltpu.SemaphoreType.DMA,
    ],
    compiler_params=_SC_PARAMS,
)
def _agg_l0(glo_hbm, ghi_hbm, eflat_hbm, padc_hbm, zero_hbm, out_lo, out_hi,
            srcs_v, dsts_v, r0, r1, r2, r3, acc_sh, sg0, sg1, sg2, sg3,
            ss0, ss1, ss2, ss3):
    c = lax.axis_index("c")
    s = lax.axis_index("s")
    rows = (r0, r1, r2, r3)
    sg = (sg0, sg1, sg2, sg3)
    ss = (ss0, ss1, ss2, ss3)
    row0 = s * RPT
    eb = s * _NBATCH2 * B

    def run(gref, outref):
        pltpu.sync_copy(zero_hbm.at[pl.ds(row0, RPT)],
                        acc_sh.at[pl.ds(row0, RPT)])

        @pl.when(s < NTILES - 1)
        def _():
            pltpu.sync_copy(eflat_hbm.at[pl.ds(eb, _NBATCH2 * B)], srcs_v)
            pltpu.sync_copy(eflat_hbm.at[pl.ds(E + eb, _NBATCH2 * B)],
                            dsts_v)

        @pl.when(s == NTILES - 1)
        def _():
            pltpu.sync_copy(eflat_hbm.at[pl.ds(eb, _LASTR2 * B)],
                            srcs_v.at[pl.ds(0, _LASTR2 * B)])
            pltpu.sync_copy(padc_hbm.at[pl.ds(0, PADB * B)],
                            srcs_v.at[pl.ds(_LASTR2 * B, PADB * B)])
            pltpu.sync_copy(eflat_hbm.at[pl.ds(E + eb, _LASTR2 * B)],
                            dsts_v.at[pl.ds(0, _LASTR2 * B)])
            pltpu.sync_copy(padc_hbm.at[pl.ds(0, PADB * B)],
                            dsts_v.at[pl.ds(_LASTR2 * B, PADB * B)])

        plsc.subcore_barrier()

        def body(j, carry):
            for p in range(4):
                i = 4 * j + p

                @pl.when(j > 0)
                def _(i=i, p=p):
                    pltpu.make_async_copy(
                        rows[p],
                        acc_sh.at[dsts_v.at[pl.ds((i - 4) * B, B)]],
                        ss[p]).wait()

                pltpu.async_copy(
                    gref.at[srcs_v.at[pl.ds(i * B, B)]], rows[p], sg[p])
            for p in range(4):
                i = 4 * j + p
                pltpu.make_async_copy(
                    gref.at[srcs_v.at[pl.ds(i * B, B)]], rows[p],
                    sg[p]).wait()
                pltpu.async_copy(
                    rows[p], acc_sh.at[dsts_v.at[pl.ds(i * B, B)]],
                    ss[p], add=True)
            return carry

        lax.fori_loop(0, _NBATCH2 // 4, body, 0)
        for p in range(4):
            pltpu.make_async_copy(
                rows[p],
                acc_sh.at[dsts_v.at[pl.ds((_NBATCH2 - 4 + p) * B, B)]],
                ss[p]).wait()
        plsc.subcore_barrier()
        pltpu.sync_copy(acc_sh.at[pl.ds(row0, RPT)],
                        outref.at[pl.ds(row0, RPT)])

    @pl.when(c == 0)
    def _():
        run(glo_hbm, out_lo)

    @pl.when(c == 1)
    def _():
        run(ghi_hbm, out_hi)


_RBLK = 1024
_GRID = NP // _RBLK


def _tc1_body(ha_ref, hb_ref, x_ref, glo_ref, ghi_ref, dinv_ref):
    i = pl.program_id(0)
    rows = lax.broadcasted_iota(jnp.int32, (_RBLK, 1), 0) + i * _RBLK
    deg = ha_ref[:, 0:1] + hb_ref[:, 0:1] + 1.0
    dinv = jnp.where(rows < N, lax.rsqrt(deg), 0.0)
    dinv_ref[...] = dinv
    g = x_ref[...] * dinv
    glo_ref[...] = g[:, :64]
    ghi_ref[...] = g[:, 64:]


def _tc1(ha, hb, x_pad):
    return pl.pallas_call(
        _tc1_body,
        grid=(_GRID,),
        in_specs=[
            pl.BlockSpec((_RBLK, 16), lambda i: (i, 0)),
            pl.BlockSpec((_RBLK, 16), lambda i: (i, 0)),
            pl.BlockSpec((_RBLK, DF), lambda i: (i, 0)),
        ],
        out_specs=[
            pl.BlockSpec((_RBLK, 64), lambda i: (i, 0)),
            pl.BlockSpec((_RBLK, 64), lambda i: (i, 0)),
            pl.BlockSpec((_RBLK, 1), lambda i: (i, 0)),
        ],
        out_shape=[
            jax.ShapeDtypeStruct((NP, 64), jnp.float32),
            jax.ShapeDtypeStruct((NP, 64), jnp.float32),
            jax.ShapeDtypeStruct((NP, 1), jnp.float32),
        ],
    )(ha, hb, x_pad)


def _tc2_body(a_ref, b_ref, glo_ref, ghi_ref, dinv_ref, w0_ref, b0_ref,
              w1_ref, q_ref):
    dinv = dinv_ref[...]
    a0 = (jnp.concatenate([a_ref[...], b_ref[...]], axis=1)
          + jnp.concatenate([glo_ref[...], ghi_ref[...]], axis=1)) * dinv
    h = jnp.dot(a0, w0_ref[...], preferred_element_type=jnp.float32)
    h = jnp.maximum(h + b0_ref[...], 0.0)
    q = jnp.dot(h, w1_ref[...], preferred_element_type=jnp.float32)
    q_ref[...] = q * dinv


def _tc2(acc_a, acc_b, glo, ghi, dinv, W0, b0r, W1p):
    return pl.pallas_call(
        _tc2_body,
        grid=(_GRID,),
        in_specs=[
            pl.BlockSpec((_RBLK, 64), lambda i: (i, 0)),
            pl.BlockSpec((_RBLK, 64), lambda i: (i, 0)),
            pl.BlockSpec((_RBLK, 64), lambda i: (i, 0)),
            pl.BlockSpec((_RBLK, 64), lambda i: (i, 0)),
            pl.BlockSpec((_RBLK, 1), lambda i: (i, 0)),
            pl.BlockSpec((DF, DF), lambda i: (0, 0)),
            pl.BlockSpec((1, DF), lambda i: (0, 0)),
            pl.BlockSpec((DF, DC), lambda i: (0, 0)),
        ],
        out_specs=pl.BlockSpec((_RBLK, DC), lambda i: (i, 0)),
        out_shape=jax.ShapeDtypeStruct((NP, DC), jnp.float32),
    )(acc_a, acc_b, glo, ghi, dinv, W0, b0r, W1p)


def _tc3_body(a_ref, b_ref, q_ref, dinv_ref, b1_ref, out_ref):
    z = ((a_ref[...] + b_ref[...] + q_ref[...]) * dinv_ref[...]
         + b1_ref[...])
    col = lax.broadcasted_iota(jnp.int32, (_RBLK, DC), 1)
    valid = col < NCLS
    zm = jnp.where(valid, z, -jnp.inf)
    m = jnp.max(zm, axis=1, keepdims=True)
    lse = jnp.log(jnp.sum(jnp.where(valid, jnp.exp(z - m), 0.0),
                          axis=1, keepdims=True))
    out_ref[...] = (z - m - lse)[:, :NCLS]


def _tc3(acc_a, acc_b, q1, dinv, b1r):
    return pl.pallas_call(
        _tc3_body,
        grid=(_GRID,),
        in_specs=[
            pl.BlockSpec((_RBLK, DC), lambda i: (i, 0)),
            pl.BlockSpec((_RBLK, DC), lambda i: (i, 0)),
            pl.BlockSpec((_RBLK, DC), lambda i: (i, 0)),
            pl.BlockSpec((_RBLK, 1), lambda i: (i, 0)),
            pl.BlockSpec((1, DC), lambda i: (0, 0)),
        ],
        out_specs=pl.BlockSpec((_RBLK, NCLS), lambda i: (i, 0)),
        out_shape=jax.ShapeDtypeStruct((NP, NCLS), jnp.float32),
    )(acc_a, acc_b, q1, dinv, b1r)


_PADC = np.int32(N) + (np.arange(PADB * B) % (NP - N)).astype(np.int32)


def kernel(x, edge_index, W0, b0, W1, b1):
    eflat = edge_index.reshape(2 * E)
    padc = jnp.asarray(_PADC)
    x_pad = jnp.zeros((NP, DF), jnp.float32).at[:N].set(x)
    ones16 = jnp.ones((B, 16), jnp.float32)
    z16 = jnp.zeros((NP, 16), jnp.float32)
    z64 = jnp.zeros((NP, DC), jnp.float32)
    W1p = jnp.zeros((DF, DC), jnp.float32).at[:, :NCLS].set(W1)
    b1r = jnp.zeros((1, DC), jnp.float32).at[0, :NCLS].set(b1)

    ha, hb = _deg(ones16, eflat, padc, z16)
    glo, ghi, dinv = _tc1(ha, hb, x_pad)
    s0lo, s0hi = _agg_l0(glo, ghi, eflat, padc, z64)
    q1 = _tc2(s0lo, s0hi, glo, ghi, dinv, W0, b0.reshape(1, DF), W1p)
    a1a, a1b = _agg64(q1, eflat, padc, z64)
    out_full = _tc3(a1a, a1b, q1, dinv, b1r)
    return out_full[:N]


# speedup vs baseline: 1.0371x; 1.0049x over previous
import functools

import numpy as np

import jax
import jax.numpy as jnp
from jax import lax
from jax.experimental import pallas as pl
from jax.experimental.pallas import tpu as pltpu
from jax.experimental.pallas import tpu_sc as plsc

N = 10000
NP = 10240
DF = 128
DC = 64
NCLS = 40
E = 320000
B = 128
NTILES = 16
NCORES = 2
EPAD = 327680
NBATCH = EPAD // B // (NTILES * NCORES)
RBATCH = E // B
PADB = EPAD // B - RBATCH
LASTR = RBATCH - (NTILES * NCORES - 1) * NBATCH
RPT = NP // NTILES

_MESH = plsc.VectorSubcoreMesh(
    core_axis_name="c", subcore_axis_name="s",
    num_cores=NCORES, num_subcores=NTILES)
_SC_PARAMS = pltpu.CompilerParams(use_tc_tiling_on_sc=False)


def _make_agg(d):
    @functools.partial(
        pl.kernel,
        out_type=(jax.ShapeDtypeStruct((NP, d), jnp.float32),
                  jax.ShapeDtypeStruct((NP, d), jnp.float32)),
        mesh=_MESH,
        scratch_types=[
            pltpu.VMEM((NBATCH * B,), jnp.int32),
            pltpu.VMEM((NBATCH * B,), jnp.int32),
            pltpu.VMEM((B, d), jnp.float32),
            pltpu.VMEM((B, d), jnp.float32),
            pltpu.VMEM((B, d), jnp.float32),
            pltpu.VMEM((B, d), jnp.float32),
            pltpu.VMEM_SHARED((NP, d), jnp.float32),
            pltpu.SemaphoreType.DMA,
            pltpu.SemaphoreType.DMA,
            pltpu.SemaphoreType.DMA,
            pltpu.SemaphoreType.DMA,
            pltpu.SemaphoreType.DMA,
            pltpu.SemaphoreType.DMA,
            pltpu.SemaphoreType.DMA,
            pltpu.SemaphoreType.DMA,
        ],
        compiler_params=_SC_PARAMS,
    )
    def agg(g_hbm, eflat_hbm, padc_hbm, zero_hbm, out_a, out_b,
            srcs_v, dsts_v, r0, r1, r2, r3, acc_sh,
            sg0, sg1, sg2, sg3, ss0, ss1, ss2, ss3):
        rows = (r0, r1, r2, r3)
        sg = (sg0, sg1, sg2, sg3)
        ss = (ss0, ss1, ss2, ss3)
        c = lax.axis_index("c")
        s = lax.axis_index("s")
        row0 = s * RPT

        pltpu.sync_copy(zero_hbm.at[pl.ds(row0, RPT)],
                        acc_sh.at[pl.ds(row0, RPT)])

        t = c * NTILES + s
        eb = t * NBATCH * B

        @pl.when(t < NTILES * NCORES - 1)
        def _():
            pltpu.sync_copy(eflat_hbm.at[pl.ds(eb, NBATCH * B)], srcs_v)
            pltpu.sync_copy(eflat_hbm.at[pl.ds(E + eb, NBATCH * B)], dsts_v)

        @pl.when(t == NTILES * NCORES - 1)
        def _():
            pltpu.sync_copy(eflat_hbm.at[pl.ds(eb, LASTR * B)],
                            srcs_v.at[pl.ds(0, LASTR * B)])
            pltpu.sync_copy(padc_hbm.at[pl.ds(0, PADB * B)],
                            srcs_v.at[pl.ds(LASTR * B, PADB * B)])
            pltpu.sync_copy(eflat_hbm.at[pl.ds(E + eb, LASTR * B)],
                            dsts_v.at[pl.ds(0, LASTR * B)])
            pltpu.sync_copy(padc_hbm.at[pl.ds(0, PADB * B)],
                            dsts_v.at[pl.ds(LASTR * B, PADB * B)])

        plsc.subcore_barrier()

        def gather(i, p):
            return pltpu.async_copy(
                g_hbm.at[srcs_v.at[pl.ds(i * B, B)]], rows[p], sg[p])

        def gather_wait(i, p):
            pltpu.make_async_copy(
                g_hbm.at[srcs_v.at[pl.ds(i * B, B)]], rows[p], sg[p]).wait()

        def scatter(i, p):
            return pltpu.async_copy(
                rows[p], acc_sh.at[dsts_v.at[pl.ds(i * B, B)]],
                ss[p], add=True)

        def scatter_wait(i, p):
            pltpu.make_async_copy(
                rows[p], acc_sh.at[dsts_v.at[pl.ds(i * B, B)]],
                ss[p]).wait()

        def body(j, carry):
            for p in range(4):
                i = 4 * j + p

                @pl.when(j > 0)
                def _(i=i, p=p):
                    scatter_wait(i - 4, p)

                gather(i, p)
            for p in range(4):
                i = 4 * j + p
                gather_wait(i, p)
                scatter(i, p)
            return carry

        lax.fori_loop(0, NBATCH // 4, body, 0)
        for p in range(4):
            scatter_wait(NBATCH - 4 + p, p)
        plsc.subcore_barrier()

        @pl.when(c == 0)
        def _():
            pltpu.sync_copy(acc_sh.at[pl.ds(row0, RPT)],
                            out_a.at[pl.ds(row0, RPT)])

        @pl.when(c == 1)
        def _():
            pltpu.sync_copy(acc_sh.at[pl.ds(row0, RPT)],
                            out_b.at[pl.ds(row0, RPT)])

    return agg


@functools.partial(
    pl.kernel,
    out_type=(jax.ShapeDtypeStruct((NP, 8), jnp.float32),
              jax.ShapeDtypeStruct((NP, 8), jnp.float32)),
    mesh=_MESH,
    scratch_types=[
        pltpu.VMEM((NBATCH * B,), jnp.int32),
        pltpu.VMEM((B, 8), jnp.float32),
        pltpu.VMEM_SHARED((NP, 8), jnp.float32),
        pltpu.SemaphoreType.DMA,
    ],
    compiler_params=_SC_PARAMS,
)
def _deg(ones_hbm, eflat_hbm, padc_hbm, zero_hbm, out_a, out_b,
         dsts_v, ones_v, acc_sh, sem):
    c = lax.axis_index("c")
    s = lax.axis_index("s")
    row0 = s * RPT
    W = 8

    pltpu.sync_copy(zero_hbm.at[pl.ds(row0, RPT)],
                    acc_sh.at[pl.ds(row0, RPT)])
    pltpu.sync_copy(ones_hbm, ones_v)

    t = c * NTILES + s
    eb = t * NBATCH * B

    @pl.when(t < NTILES * NCORES - 1)
    def _():
        pltpu.sync_copy(eflat_hbm.at[pl.ds(E + eb, NBATCH * B)], dsts_v)

    @pl.when(t == NTILES * NCORES - 1)
    def _():
        pltpu.sync_copy(eflat_hbm.at[pl.ds(E + eb, LASTR * B)],
                        dsts_v.at[pl.ds(0, LASTR * B)])
        pltpu.sync_copy(padc_hbm.at[pl.ds(0, PADB * B)],
                        dsts_v.at[pl.ds(LASTR * B, PADB * B)])

    plsc.subcore_barrier()

    def body(i, carry):
        pltpu.async_copy(ones_v, acc_sh.at[dsts_v.at[pl.ds(i * B, B)]],
                         sem, add=True)

        @pl.when(i >= W)
        def _():
            pltpu.make_async_copy(
                ones_v, acc_sh.at[dsts_v.at[pl.ds((i - W) * B, B)]],
                sem).wait()
        return carry

    lax.fori_loop(0, NBATCH, body, 0)

    def drain(i, carry):
        pltpu.make_async_copy(
            ones_v, acc_sh.at[dsts_v.at[pl.ds((NBATCH - W + i) * B, B)]],
            sem).wait()
        return carry

    lax.fori_loop(0, W, drain, 0)
    plsc.subcore_barrier()

    @pl.when(c == 0)
    def _():
        pltpu.sync_copy(acc_sh.at[pl.ds(row0, RPT)],
                        out_a.at[pl.ds(row0, RPT)])

    @pl.when(c == 1)
    def _():
        pltpu.sync_copy(acc_sh.at[pl.ds(row0, RPT)],
                        out_b.at[pl.ds(row0, RPT)])


_agg64 = _make_agg(DC)

_NBATCH2 = EPAD // B // NTILES
_LASTR2 = RBATCH - (NTILES - 1) * _NBATCH2


@functools.partial(
    pl.kernel,
    out_type=(jax.ShapeDtypeStruct((NP, 64), jnp.float32),
              jax.ShapeDtypeStruct((NP, 64), jnp.float32)),
    mesh=_MESH,
    scratch_types=[
        pltpu.VMEM((_NBATCH2 * B,), jnp.int32),
        pltpu.VMEM((_NBATCH2 * B,), jnp.int32),
        pltpu.VMEM((B, 64), jnp.float32),
        pltpu.VMEM((B, 64), jnp.float32),
        pltpu.VMEM((B, 64), jnp.float32),
        pltpu.VMEM((B, 64), jnp.float32),
        pltpu.VMEM_SHARED((NP, 64), jnp.float32),
        pltpu.SemaphoreType.DMA,
        pltpu.SemaphoreType.DMA,
        pltpu.SemaphoreType.DMA,
        pltpu.SemaphoreType.DMA,
        pltpu.SemaphoreType.DMA,
        pltpu.SemaphoreType.DMA,
        pltpu.SemaphoreType.DMA,
        pltpu.SemaphoreType.DMA,
    ],
    compiler_params=_SC_PARAMS,
)
def _agg_l0(glo_hbm, ghi_hbm, eflat_hbm, padc_hbm, zero_hbm, out_lo, out_hi,
            srcs_v, dsts_v, r0, r1, r2, r3, acc_sh, sg0, sg1, sg2, sg3,
            ss0, ss1, ss2, ss3):
    c = lax.axis_index("c")
    s = lax.axis_index("s")
    rows = (r0, r1, r2, r3)
    sg = (sg0, sg1, sg2, sg3)
    ss = (ss0, ss1, ss2, ss3)
    row0 = s * RPT
    eb = s * _NBATCH2 * B

    def run(gref, outref):
        pltpu.sync_copy(zero_hbm.at[pl.ds(row0, RPT)],
                        acc_sh.at[pl.ds(row0, RPT)])

        @pl.when(s < NTILES - 1)
        def _():
            pltpu.sync_copy(eflat_hbm.at[pl.ds(eb, _NBATCH2 * B)], srcs_v)
            pltpu.sync_copy(eflat_hbm.at[pl.ds(E + eb, _NBATCH2 * B)],
                            dsts_v)

        @pl.when(s == NTILES - 1)
        def _():
            pltpu.sync_copy(eflat_hbm.at[pl.ds(eb, _LASTR2 * B)],
                            srcs_v.at[pl.ds(0, _LASTR2 * B)])
            pltpu.sync_copy(padc_hbm.at[pl.ds(0, PADB * B)],
                            srcs_v.at[pl.ds(_LASTR2 * B, PADB * B)])
            pltpu.sync_copy(eflat_hbm.at[pl.ds(E + eb, _LASTR2 * B)],
                            dsts_v.at[pl.ds(0, _LASTR2 * B)])
            pltpu.sync_copy(padc_hbm.at[pl.ds(0, PADB * B)],
                            dsts_v.at[pl.ds(_LASTR2 * B, PADB * B)])

        plsc.subcore_barrier()

        def body(j, carry):
            for p in range(4):
                i = 4 * j + p

                @pl.when(j > 0)
                def _(i=i, p=p):
                    pltpu.make_async_copy(
                        rows[p],
                        acc_sh.at[dsts_v.at[pl.ds((i - 4) * B, B)]],
                        ss[p]).wait()

                pltpu.async_copy(
                    gref.at[srcs_v.at[pl.ds(i * B, B)]], rows[p], sg[p])
            for p in range(4):
                i = 4 * j + p
                pltpu.make_async_copy(
                    gref.at[srcs_v.at[pl.ds(i * B, B)]], rows[p],
                    sg[p]).wait()
                pltpu.async_copy(
                    rows[p], acc_sh.at[dsts_v.at[pl.ds(i * B, B)]],
                    ss[p], add=True)
            return carry

        lax.fori_loop(0, _NBATCH2 // 4, body, 0)
        for p in range(4):
            pltpu.make_async_copy(
                rows[p],
                acc_sh.at[dsts_v.at[pl.ds((_NBATCH2 - 4 + p) * B, B)]],
                ss[p]).wait()
        plsc.subcore_barrier()
        pltpu.sync_copy(acc_sh.at[pl.ds(row0, RPT)],
                        outref.at[pl.ds(row0, RPT)])

    @pl.when(c == 0)
    def _():
        run(glo_hbm, out_lo)

    @pl.when(c == 1)
    def _():
        run(ghi_hbm, out_hi)


_RBLK = 1024
_GRID = NP // _RBLK


def _tc1_body(ha_ref, hb_ref, x_ref, glo_ref, ghi_ref, dinv_ref):
    i = pl.program_id(0)
    rows = lax.broadcasted_iota(jnp.int32, (_RBLK, 1), 0) + i * _RBLK
    deg = ha_ref[:, 0:1] + hb_ref[:, 0:1] + 1.0
    dinv = jnp.where(rows < N, lax.rsqrt(deg), 0.0)
    dinv_ref[...] = dinv
    g = x_ref[...] * dinv
    glo_ref[...] = g[:, :64]
    ghi_ref[...] = g[:, 64:]


def _tc1(ha, hb, x_pad):
    return pl.pallas_call(
        _tc1_body,
        grid=(_GRID,),
        in_specs=[
            pl.BlockSpec((_RBLK, 8), lambda i: (i, 0)),
            pl.BlockSpec((_RBLK, 8), lambda i: (i, 0)),
            pl.BlockSpec((_RBLK, DF), lambda i: (i, 0)),
        ],
        out_specs=[
            pl.BlockSpec((_RBLK, 64), lambda i: (i, 0)),
            pl.BlockSpec((_RBLK, 64), lambda i: (i, 0)),
            pl.BlockSpec((_RBLK, 1), lambda i: (i, 0)),
        ],
        out_shape=[
            jax.ShapeDtypeStruct((NP, 64), jnp.float32),
            jax.ShapeDtypeStruct((NP, 64), jnp.float32),
            jax.ShapeDtypeStruct((NP, 1), jnp.float32),
        ],
    )(ha, hb, x_pad)


def _tc2_body(a_ref, b_ref, glo_ref, ghi_ref, dinv_ref, w0_ref, b0_ref,
              w1_ref, q_ref):
    dinv = dinv_ref[...]
    a0 = (jnp.concatenate([a_ref[...], b_ref[...]], axis=1)
          + jnp.concatenate([glo_ref[...], ghi_ref[...]], axis=1)) * dinv
    h = jnp.dot(a0, w0_ref[...], preferred_element_type=jnp.float32)
    h = jnp.maximum(h + b0_ref[...], 0.0)
    q = jnp.dot(h, w1_ref[...], preferred_element_type=jnp.float32)
    q_ref[...] = q * dinv


def _tc2(acc_a, acc_b, glo, ghi, dinv, W0, b0r, W1p):
    return pl.pallas_call(
        _tc2_body,
        grid=(_GRID,),
        in_specs=[
            pl.BlockSpec((_RBLK, 64), lambda i: (i, 0)),
            pl.BlockSpec((_RBLK, 64), lambda i: (i, 0)),
            pl.BlockSpec((_RBLK, 64), lambda i: (i, 0)),
            pl.BlockSpec((_RBLK, 64), lambda i: (i, 0)),
            pl.BlockSpec((_RBLK, 1), lambda i: (i, 0)),
            pl.BlockSpec((DF, DF), lambda i: (0, 0)),
            pl.BlockSpec((1, DF), lambda i: (0, 0)),
            pl.BlockSpec((DF, DC), lambda i: (0, 0)),
        ],
        out_specs=pl.BlockSpec((_RBLK, DC), lambda i: (i, 0)),
        out_shape=jax.ShapeDtypeStruct((NP, DC), jnp.float32),
    )(acc_a, acc_b, glo, ghi, dinv, W0, b0r, W1p)


def _tc3_body(a_ref, b_ref, q_ref, dinv_ref, b1_ref, out_ref):
    z = ((a_ref[...] + b_ref[...] + q_ref[...]) * dinv_ref[...]
         + b1_ref[...])
    col = lax.broadcasted_iota(jnp.int32, (_RBLK, DC), 1)
    valid = col < NCLS
    zm = jnp.where(valid, z, -jnp.inf)
    m = jnp.max(zm, axis=1, keepdims=True)
    lse = jnp.log(jnp.sum(jnp.where(valid, jnp.exp(z - m), 0.0),
                          axis=1, keepdims=True))
    out_ref[...] = (z - m - lse)[:, :NCLS]


def _tc3(acc_a, acc_b, q1, dinv, b1r):
    return pl.pallas_call(
        _tc3_body,
        grid=(_GRID,),
        in_specs=[
            pl.BlockSpec((_RBLK, DC), lambda i: (i, 0)),
            pl.BlockSpec((_RBLK, DC), lambda i: (i, 0)),
            pl.BlockSpec((_RBLK, DC), lambda i: (i, 0)),
            pl.BlockSpec((_RBLK, 1), lambda i: (i, 0)),
            pl.BlockSpec((1, DC), lambda i: (0, 0)),
        ],
        out_specs=pl.BlockSpec((_RBLK, NCLS), lambda i: (i, 0)),
        out_shape=jax.ShapeDtypeStruct((NP, NCLS), jnp.float32),
    )(acc_a, acc_b, q1, dinv, b1r)


_PADC = np.int32(N) + (np.arange(PADB * B) % (NP - N)).astype(np.int32)


def kernel(x, edge_index, W0, b0, W1, b1):
    eflat = edge_index.reshape(2 * E)
    padc = jnp.asarray(_PADC)
    x_pad = jnp.zeros((NP, DF), jnp.float32).at[:N].set(x)
    ones16 = jnp.ones((B, 8), jnp.float32)
    z16 = jnp.zeros((NP, 8), jnp.float32)
    z64 = jnp.zeros((NP, DC), jnp.float32)
    W1p = jnp.zeros((DF, DC), jnp.float32).at[:, :NCLS].set(W1)
    b1r = jnp.zeros((1, DC), jnp.float32).at[0, :NCLS].set(b1)

    ha, hb = _deg(ones16, eflat, padc, z16)
    glo, ghi, dinv = _tc1(ha, hb, x_pad)
    s0lo, s0hi = _agg_l0(glo, ghi, eflat, padc, z64)
    q1 = _tc2(s0lo, s0hi, glo, ghi, dinv, W0, b0.reshape(1, DF), W1p)
    a1a, a1b = _agg64(q1, eflat, padc, z64)
    out_full = _tc3(a1a, a1b, q1, dinv, b1r)
    return out_full[:N]
